# baseline (speedup 1.0000x reference)
import jax
import jax.numpy as jnp
from jax import lax
from jax.experimental import pallas as pl
from jax.experimental.pallas import tpu as pltpu
from jax.experimental.pallas import tpu_sc as plsc

_N = 10000
_E = 320000
_DIN = 128
_DH = 16
_DOUT = 128

_NC = 1
_NS = 16
_NW = _NC * _NS
_EP = 327680
_CR = 1024
_EPW = _EP // _NW
_NCHUNK = _EPW // _CR
_DUMP = _N
_ACC_ROWS = 10240
_TBL_ROWS = _N + 16
_ZSL = _ACC_ROWS // _NS

_f32 = jnp.float32


def _deg_body(zeros_hbm, ones_hbm, src_hbm, dst_hbm, out_hbm, idxs_all,
              idxd_all, ones_v, acc_o, acc_i, sem_o, sem_i):
  cid = lax.axis_index("c")
  sid = lax.axis_index("s")
  wid = cid * _NS + sid

  pltpu.sync_copy(zeros_hbm.at[pl.ds(sid * _ZSL, _ZSL)],
                  acc_o.at[pl.ds(sid * _ZSL, _ZSL)])
  pltpu.sync_copy(zeros_hbm.at[pl.ds(sid * _ZSL, _ZSL)],
                  acc_i.at[pl.ds(sid * _ZSL, _ZSL)])
  pltpu.sync_copy(ones_hbm, ones_v)
  base = wid * _NCHUNK
  pltpu.sync_copy(src_hbm.at[pl.ds(base, _NCHUNK)], idxs_all)
  pltpu.sync_copy(dst_hbm.at[pl.ds(base, _NCHUNK)], idxd_all)
  plsc.subcore_barrier()

  def _chunk(c, carry):
    io = idxs_all.at[c]
    ii = idxd_all.at[c]
    pltpu.async_copy(ones_v, acc_o.at[io], sem_o, add=True)
    pltpu.async_copy(ones_v, acc_i.at[ii], sem_i, add=True)
    pltpu.make_async_copy(ones_v, acc_o.at[io], sem_o).wait()
    pltpu.make_async_copy(ones_v, acc_i.at[ii], sem_i).wait()
    return carry

  lax.fori_loop(0, _NCHUNK, _chunk, 0)
  plsc.subcore_barrier()

  pltpu.sync_copy(acc_o.at[pl.ds(sid * _ZSL, _ZSL)],
                  out_hbm.at[cid, 0, pl.ds(sid * _ZSL, _ZSL)])
  pltpu.sync_copy(acc_i.at[pl.ds(sid * _ZSL, _ZSL)],
                  out_hbm.at[cid, 1, pl.ds(sid * _ZSL, _ZSL)])


_deg_call = pl.kernel(
    _deg_body,
    out_type=jax.ShapeDtypeStruct((_NC, 2, _ACC_ROWS, _DH), _f32),
    mesh=plsc.VectorSubcoreMesh(core_axis_name="c", subcore_axis_name="s", num_cores=_NC),
    scratch_types=[
        pltpu.VMEM((_NCHUNK, _CR), jnp.int32),
        pltpu.VMEM((_NCHUNK, _CR), jnp.int32),
        pltpu.VMEM((_CR, _DH), _f32),
        pltpu.VMEM_SHARED((_ACC_ROWS, _DH), _f32),
        pltpu.VMEM_SHARED((_ACC_ROWS, _DH), _f32),
        pltpu.SemaphoreType.DMA,
        pltpu.SemaphoreType.DMA,
    ],
    compiler_params=pltpu.CompilerParams(use_tc_tiling_on_sc=False),
)


def _agg_body(zeros_hbm, h_hbm, src_hbm, dst_hbm, out_hbm, idxs_all, idxd_all,
              r_a, r_b, acc, sem_a, sem_b):
  cid = lax.axis_index("c")
  sid = lax.axis_index("s")
  wid = cid * _NS + sid

  pltpu.sync_copy(zeros_hbm.at[pl.ds(sid * _ZSL, _ZSL)],
                  acc.at[pl.ds(sid * _ZSL, _ZSL)])
  base = wid * _NCHUNK
  pltpu.sync_copy(src_hbm.at[pl.ds(base, _NCHUNK)], idxs_all)
  pltpu.sync_copy(dst_hbm.at[pl.ds(base, _NCHUNK)], idxd_all)
  plsc.subcore_barrier()

  def _fire(chunk, rows, sem):
    c = jnp.minimum(chunk, _NCHUNK - 1)
    pltpu.async_copy(h_hbm.at[idxs_all.at[c]], rows, sem)

  def _drain(rows, sem):
    pltpu.make_async_copy(h_hbm.at[pl.ds(0, _CR)], rows, sem).wait()

  def _scatter(chunk, rows):
    pltpu.sync_copy(rows, acc.at[idxd_all.at[chunk]], add=True)

  _fire(0, r_a, sem_a)

  def _pair(p, carry):
    ca = 2 * p
    cb = ca + 1
    _fire(cb, r_b, sem_b)
    _drain(r_a, sem_a)
    _scatter(ca, r_a)
    _fire(ca + 2, r_a, sem_a)
    _drain(r_b, sem_b)
    _scatter(cb, r_b)
    return carry

  lax.fori_loop(0, _NCHUNK // 2, _pair, 0)
  _drain(r_a, sem_a)
  plsc.subcore_barrier()

  pltpu.sync_copy(acc.at[pl.ds(sid * _ZSL, _ZSL)],
                  out_hbm.at[cid, pl.ds(sid * _ZSL, _ZSL)])


_agg_call = pl.kernel(
    _agg_body,
    out_type=jax.ShapeDtypeStruct((_NC, _ACC_ROWS, _DH), _f32),
    mesh=plsc.VectorSubcoreMesh(core_axis_name="c", subcore_axis_name="s", num_cores=_NC),
    scratch_types=[
        pltpu.VMEM((_NCHUNK, _CR), jnp.int32),
        pltpu.VMEM((_NCHUNK, _CR), jnp.int32),
        pltpu.VMEM((_CR, _DH), _f32),
        pltpu.VMEM((_CR, _DH), _f32),
        pltpu.VMEM_SHARED((_ACC_ROWS, _DH), _f32),
        pltpu.SemaphoreType.DMA,
        pltpu.SemaphoreType.DMA,
    ],
    compiler_params=pltpu.CompilerParams(use_tc_tiling_on_sc=False),
)


_GRID = 10
_BR = _N // _GRID


def _norms(deg_ref):
  deg_o = sum(deg_ref[c, 0] for c in range(_NC))
  deg_i = sum(deg_ref[c, 1] for c in range(_NC))
  ns = jnp.where(deg_o > 0, lax.rsqrt(jnp.maximum(deg_o, 1.0)), 1.0)
  nd = jnp.where(deg_i > 0, lax.rsqrt(jnp.maximum(deg_i, 1.0)), 1.0)
  return ns, nd


def _l1_body(x_ref, w_ref, deg_ref, o_ref):
  ns, _ = _norms(deg_ref)
  o_ref[...] = jnp.dot(x_ref[...], w_ref[...], preferred_element_type=_f32) * ns


_l1_call = pl.pallas_call(
    _l1_body,
    grid=(_GRID,),
    in_specs=[
        pl.BlockSpec((_BR, _DIN), lambda i: (i, 0)),
        pl.BlockSpec((_DIN, _DH), lambda i: (0, 0)),
        pl.BlockSpec((_NC, 2, _BR, _DH), lambda i: (0, 0, i, 0)),
    ],
    out_specs=pl.BlockSpec((_BR, _DH), lambda i: (i, 0)),
    out_shape=jax.ShapeDtypeStruct((_TBL_ROWS, _DH), _f32),
)


def _mid_body(a_ref, deg_ref, b_ref, o_ref):
  a = sum(a_ref[c] for c in range(_NC))
  ns, nd = _norms(deg_ref)
  h = jnp.maximum(a * nd + b_ref[...], 0.0)
  o_ref[...] = h * ns


_mid_call = pl.pallas_call(
    _mid_body,
    grid=(_GRID,),
    in_specs=[
        pl.BlockSpec((_NC, _BR, _DH), lambda i: (0, i, 0)),
        pl.BlockSpec((_NC, 2, _BR, _DH), lambda i: (0, 0, i, 0)),
        pl.BlockSpec((1, _DH), lambda i: (0, 0)),
    ],
    out_specs=pl.BlockSpec((_BR, _DH), lambda i: (i, 0)),
    out_shape=jax.ShapeDtypeStruct((_TBL_ROWS, _DH), _f32),
)


def _fin_body(a_ref, deg_ref, w_ref, b_ref, o_ref):
  a = sum(a_ref[c] for c in range(_NC))
  _, nd = _norms(deg_ref)
  o_ref[...] = (
      jnp.dot(a * nd, w_ref[...], preferred_element_type=_f32) + b_ref[...])


_fin_call = pl.pallas_call(
    _fin_body,
    grid=(_GRID,),
    in_specs=[
        pl.BlockSpec((_NC, _BR, _DH), lambda i: (0, i, 0)),
        pl.BlockSpec((_NC, 2, _BR, _DH), lambda i: (0, 0, i, 0)),
        pl.BlockSpec((_DH, _DOUT), lambda i: (0, 0)),
        pl.BlockSpec((1, _DOUT), lambda i: (0, 0)),
    ],
    out_specs=pl.BlockSpec((_BR, _DOUT), lambda i: (i, 0)),
    out_shape=jax.ShapeDtypeStruct((_N, _DOUT), _f32),
)


@jax.jit
def kernel(x, edge_index, W1, b1, W2, b2):
  src = edge_index[0].astype(jnp.int32)
  dst = edge_index[1].astype(jnp.int32)
  pad = _EP - _E
  padv = jnp.full((pad,), _DUMP, jnp.int32)
  srcp = jnp.concatenate([src, padv]).reshape(_EP // _CR, _CR)
  dstp = jnp.concatenate([dst, padv]).reshape(_EP // _CR, _CR)

  zeros_acc = jnp.zeros((_ACC_ROWS, _DH), _f32)
  ones_cr = jnp.ones((_CR, _DH), _f32)

  degp = _deg_call(zeros_acc, ones_cr, srcp, dstp)
  h1t = _l1_call(x, W1, degp)
  a1p = _agg_call(zeros_acc, h1t, srcp, dstp)
  h2t = _mid_call(a1p, degp, b1.reshape(1, _DH))
  a2p = _agg_call(zeros_acc, h2t, srcp, dstp)
  return _fin_call(a2p, degp, W2, b2.reshape(1, _DOUT))


# speedup vs baseline: 1.0570x; 1.0570x over previous
import jax
import jax.numpy as jnp
from jax import lax
from jax.experimental import pallas as pl
from jax.experimental.pallas import tpu as pltpu
from jax.experimental.pallas import tpu_sc as plsc

_N = 10000
_E = 320000
_DIN = 128
_DH = 16
_DOUT = 128

_NC = 2
_NS = 16
_NW = _NC * _NS
_EP = 327680
_CR = 1024
_TOTC = _EP // _CR
_NCHUNK = _TOTC // _NW
_K0 = 14
_K1 = 6
_KMAX = max(_K0, _K1)
_DUMP = _N
_ACC_ROWS = 10240
_TBL_ROWS = _N + 16
_ZSL = _ACC_ROWS // _NS

_f32 = jnp.float32


def _deg_body(zeros_hbm, ones_hbm, src_hbm, dst_hbm, out_hbm, idxs_all,
              idxd_all, ones_v, acc_o, acc_i, sem_o, sem_i):
  cid = lax.axis_index("c")
  sid = lax.axis_index("s")
  wid = cid * _NS + sid

  pltpu.sync_copy(zeros_hbm.at[pl.ds(sid * _ZSL, _ZSL)],
                  acc_o.at[pl.ds(sid * _ZSL, _ZSL)])
  pltpu.sync_copy(zeros_hbm.at[pl.ds(sid * _ZSL, _ZSL)],
                  acc_i.at[pl.ds(sid * _ZSL, _ZSL)])
  pltpu.sync_copy(ones_hbm, ones_v)
  base = wid * _NCHUNK
  pltpu.sync_copy(src_hbm.at[pl.ds(base, _NCHUNK)], idxs_all)
  pltpu.sync_copy(dst_hbm.at[pl.ds(base, _NCHUNK)], idxd_all)
  plsc.subcore_barrier()

  def _chunk(c, carry):
    io = idxs_all.at[c]
    ii = idxd_all.at[c]
    pltpu.async_copy(ones_v, acc_o.at[io], sem_o, add=True)
    pltpu.async_copy(ones_v, acc_i.at[ii], sem_i, add=True)
    pltpu.make_async_copy(ones_v, acc_o.at[io], sem_o).wait()
    pltpu.make_async_copy(ones_v, acc_i.at[ii], sem_i).wait()
    return carry

  lax.fori_loop(0, _NCHUNK, _chunk, 0)
  plsc.subcore_barrier()

  pltpu.sync_copy(acc_o.at[pl.ds(sid * _ZSL, _ZSL)],
                  out_hbm.at[cid, 0, pl.ds(sid * _ZSL, _ZSL)])
  pltpu.sync_copy(acc_i.at[pl.ds(sid * _ZSL, _ZSL)],
                  out_hbm.at[cid, 1, pl.ds(sid * _ZSL, _ZSL)])


_deg_call = pl.kernel(
    _deg_body,
    out_type=jax.ShapeDtypeStruct((_NC, 2, _ACC_ROWS, _DH), _f32),
    mesh=plsc.VectorSubcoreMesh(core_axis_name="c", subcore_axis_name="s", num_cores=_NC),
    scratch_types=[
        pltpu.VMEM((_NCHUNK, _CR), jnp.int32),
        pltpu.VMEM((_NCHUNK, _CR), jnp.int32),
        pltpu.VMEM((_CR, _DH), _f32),
        pltpu.VMEM_SHARED((_ACC_ROWS, _DH), _f32),
        pltpu.VMEM_SHARED((_ACC_ROWS, _DH), _f32),
        pltpu.SemaphoreType.DMA,
        pltpu.SemaphoreType.DMA,
    ],
    compiler_params=pltpu.CompilerParams(use_tc_tiling_on_sc=False),
)


def _agg_body(zeros_hbm, h_hbm, src_hbm, dst_hbm, out_hbm, idxs_all, idxd_all,
              r_a, r_b, acc, sem_a, sem_b):
  cid = lax.axis_index("c")
  sid = lax.axis_index("s")
  wid = cid * _NS + sid

  pltpu.sync_copy(zeros_hbm.at[pl.ds(sid * _ZSL, _ZSL)],
                  acc.at[pl.ds(sid * _ZSL, _ZSL)])
  nchunk = jnp.where(cid == 0, _K0, _K1)
  base = jnp.where(cid == 0, sid * _K0, _NS * _K0 + sid * _K1)
  start = jnp.minimum(base, _TOTC - _KMAX)
  off = base - start
  pltpu.sync_copy(src_hbm.at[pl.ds(start, _KMAX)], idxs_all)
  pltpu.sync_copy(dst_hbm.at[pl.ds(start, _KMAX)], idxd_all)
  plsc.subcore_barrier()

  def _fire(chunk, rows, sem):
    c = jnp.minimum(chunk, nchunk - 1)
    pltpu.async_copy(h_hbm.at[idxs_all.at[off + c]], rows, sem)

  def _drain(rows, sem):
    pltpu.make_async_copy(h_hbm.at[pl.ds(0, _CR)], rows, sem).wait()

  def _scatter(chunk, rows):
    pltpu.sync_copy(rows, acc.at[idxd_all.at[off + chunk]], add=True)

  _fire(0, r_a, sem_a)

  def _pair(p, carry):
    ca = 2 * p
    cb = ca + 1
    _fire(cb, r_b, sem_b)
    _drain(r_a, sem_a)
    _scatter(ca, r_a)
    _fire(ca + 2, r_a, sem_a)
    _drain(r_b, sem_b)
    _scatter(cb, r_b)
    return carry

  lax.fori_loop(0, nchunk // 2, _pair, 0)
  _drain(r_a, sem_a)
  plsc.subcore_barrier()

  pltpu.sync_copy(acc.at[pl.ds(sid * _ZSL, _ZSL)],
                  out_hbm.at[cid, pl.ds(sid * _ZSL, _ZSL)])


_agg_call = pl.kernel(
    _agg_body,
    out_type=jax.ShapeDtypeStruct((_NC, _ACC_ROWS, _DH), _f32),
    mesh=plsc.VectorSubcoreMesh(core_axis_name="c", subcore_axis_name="s", num_cores=_NC),
    scratch_types=[
        pltpu.VMEM((_KMAX, _CR), jnp.int32),
        pltpu.VMEM((_KMAX, _CR), jnp.int32),
        pltpu.VMEM((_CR, _DH), _f32),
        pltpu.VMEM((_CR, _DH), _f32),
        pltpu.VMEM_SHARED((_ACC_ROWS, _DH), _f32),
        pltpu.SemaphoreType.DMA,
        pltpu.SemaphoreType.DMA,
    ],
    compiler_params=pltpu.CompilerParams(use_tc_tiling_on_sc=False),
)


_GRID = 10
_BR = _N // _GRID


def _norms(deg_ref):
  deg_o = sum(deg_ref[c, 0] for c in range(_NC))
  deg_i = sum(deg_ref[c, 1] for c in range(_NC))
  ns = jnp.where(deg_o > 0, lax.rsqrt(jnp.maximum(deg_o, 1.0)), 1.0)
  nd = jnp.where(deg_i > 0, lax.rsqrt(jnp.maximum(deg_i, 1.0)), 1.0)
  return ns, nd


def _l1_body(x_ref, w_ref, deg_ref, o_ref):
  ns, _ = _norms(deg_ref)
  o_ref[...] = jnp.dot(x_ref[...], w_ref[...], preferred_element_type=_f32) * ns


_l1_call = pl.pallas_call(
    _l1_body,
    grid=(_GRID,),
    in_specs=[
        pl.BlockSpec((_BR, _DIN), lambda i: (i, 0)),
        pl.BlockSpec((_DIN, _DH), lambda i: (0, 0)),
        pl.BlockSpec((_NC, 2, _BR, _DH), lambda i: (0, 0, i, 0)),
    ],
    out_specs=pl.BlockSpec((_BR, _DH), lambda i: (i, 0)),
    out_shape=jax.ShapeDtypeStruct((_TBL_ROWS, _DH), _f32),
)


def _mid_body(a_ref, deg_ref, b_ref, o_ref):
  a = sum(a_ref[c] for c in range(_NC))
  ns, nd = _norms(deg_ref)
  h = jnp.maximum(a * nd + b_ref[...], 0.0)
  o_ref[...] = h * ns


_mid_call = pl.pallas_call(
    _mid_body,
    grid=(_GRID,),
    in_specs=[
        pl.BlockSpec((_NC, _BR, _DH), lambda i: (0, i, 0)),
        pl.BlockSpec((_NC, 2, _BR, _DH), lambda i: (0, 0, i, 0)),
        pl.BlockSpec((1, _DH), lambda i: (0, 0)),
    ],
    out_specs=pl.BlockSpec((_BR, _DH), lambda i: (i, 0)),
    out_shape=jax.ShapeDtypeStruct((_TBL_ROWS, _DH), _f32),
)


def _fin_body(a_ref, deg_ref, w_ref, b_ref, o_ref):
  a = sum(a_ref[c] for c in range(_NC))
  _, nd = _norms(deg_ref)
  o_ref[...] = (
      jnp.dot(a * nd, w_ref[...], preferred_element_type=_f32) + b_ref[...])


_fin_call = pl.pallas_call(
    _fin_body,
    grid=(_GRID,),
    in_specs=[
        pl.BlockSpec((_NC, _BR, _DH), lambda i: (0, i, 0)),
        pl.BlockSpec((_NC, 2, _BR, _DH), lambda i: (0, 0, i, 0)),
        pl.BlockSpec((_DH, _DOUT), lambda i: (0, 0)),
        pl.BlockSpec((1, _DOUT), lambda i: (0, 0)),
    ],
    out_specs=pl.BlockSpec((_BR, _DOUT), lambda i: (i, 0)),
    out_shape=jax.ShapeDtypeStruct((_N, _DOUT), _f32),
)


@jax.jit
def kernel(x, edge_index, W1, b1, W2, b2):
  src = edge_index[0].astype(jnp.int32)
  dst = edge_index[1].astype(jnp.int32)
  pad = _EP - _E
  padv = jnp.full((pad,), _DUMP, jnp.int32)
  srcp = jnp.concatenate([src, padv]).reshape(_EP // _CR, _CR)
  dstp = jnp.concatenate([dst, padv]).reshape(_EP // _CR, _CR)

  zeros_acc = jnp.zeros((_ACC_ROWS, _DH), _f32)
  ones_cr = jnp.ones((_CR, _DH), _f32)

  degp = _deg_call(zeros_acc, ones_cr, srcp, dstp)
  h1t = _l1_call(x, W1, degp)
  a1p = _agg_call(zeros_acc, h1t, srcp, dstp)
  h2t = _mid_call(a1p, degp, b1.reshape(1, _DH))
  a2p = _agg_call(zeros_acc, h2t, srcp, dstp)
  return _fin_call(a2p, degp, W2, b2.reshape(1, _DOUT))


# speedup vs baseline: 1.0648x; 1.0074x over previous
import jax
import jax.numpy as jnp
from jax import lax
from jax.experimental import pallas as pl
from jax.experimental.pallas import tpu as pltpu
from jax.experimental.pallas import tpu_sc as plsc

_N = 10000
_E = 320000
_DIN = 128
_DH = 16
_DOUT = 128

_NC = 2
_NS = 16
_NW = _NC * _NS
_EP = 327680
_CR = 1024
_TOTC = _EP // _CR
_NCHUNK = _TOTC // _NW
_K0 = 6
_K1 = 14
_KMAX = max(_K0, _K1)
_DUMP = _N
_ACC_ROWS = 10240
_TBL_ROWS = _N + 16
_ZSL = _ACC_ROWS // _NS

_f32 = jnp.float32


def _deg_body(zeros_hbm, ones_hbm, src_hbm, dst_hbm, out_hbm, idxs_all,
              idxd_all, ones_v, acc_o, acc_i, sem_o, sem_i):
  cid = lax.axis_index("c")
  sid = lax.axis_index("s")
  wid = cid * _NS + sid

  pltpu.sync_copy(zeros_hbm.at[pl.ds(sid * _ZSL, _ZSL)],
                  acc_o.at[pl.ds(sid * _ZSL, _ZSL)])
  pltpu.sync_copy(zeros_hbm.at[pl.ds(sid * _ZSL, _ZSL)],
                  acc_i.at[pl.ds(sid * _ZSL, _ZSL)])
  pltpu.sync_copy(ones_hbm, ones_v)
  base = wid * _NCHUNK
  pltpu.sync_copy(src_hbm.at[pl.ds(base, _NCHUNK)], idxs_all)
  pltpu.sync_copy(dst_hbm.at[pl.ds(base, _NCHUNK)], idxd_all)
  plsc.subcore_barrier()

  def _chunk(c, carry):
    io = idxs_all.at[c]
    ii = idxd_all.at[c]
    pltpu.async_copy(ones_v, acc_o.at[io], sem_o, add=True)
    pltpu.async_copy(ones_v, acc_i.at[ii], sem_i, add=True)
    pltpu.make_async_copy(ones_v, acc_o.at[io], sem_o).wait()
    pltpu.make_async_copy(ones_v, acc_i.at[ii], sem_i).wait()
    return carry

  lax.fori_loop(0, _NCHUNK, _chunk, 0)
  plsc.subcore_barrier()

  pltpu.sync_copy(acc_o.at[pl.ds(sid * _ZSL, _ZSL)],
                  out_hbm.at[cid, 0, pl.ds(sid * _ZSL, _ZSL)])
  pltpu.sync_copy(acc_i.at[pl.ds(sid * _ZSL, _ZSL)],
                  out_hbm.at[cid, 1, pl.ds(sid * _ZSL, _ZSL)])


_deg_call = pl.kernel(
    _deg_body,
    out_type=jax.ShapeDtypeStruct((_NC, 2, _ACC_ROWS, _DH), _f32),
    mesh=plsc.VectorSubcoreMesh(core_axis_name="c", subcore_axis_name="s", num_cores=_NC),
    scratch_types=[
        pltpu.VMEM((_NCHUNK, _CR), jnp.int32),
        pltpu.VMEM((_NCHUNK, _CR), jnp.int32),
        pltpu.VMEM((_CR, _DH), _f32),
        pltpu.VMEM_SHARED((_ACC_ROWS, _DH), _f32),
        pltpu.VMEM_SHARED((_ACC_ROWS, _DH), _f32),
        pltpu.SemaphoreType.DMA,
        pltpu.SemaphoreType.DMA,
    ],
    compiler_params=pltpu.CompilerParams(use_tc_tiling_on_sc=False),
)


def _agg_body(zeros_hbm, h_hbm, src_hbm, dst_hbm, out_hbm, idxs_all, idxd_all,
              r_a, r_b, acc, sem_a, sem_b):
  cid = lax.axis_index("c")
  sid = lax.axis_index("s")
  wid = cid * _NS + sid

  pltpu.sync_copy(zeros_hbm.at[pl.ds(sid * _ZSL, _ZSL)],
                  acc.at[pl.ds(sid * _ZSL, _ZSL)])
  nchunk = jnp.where(cid == 0, _K0, _K1)
  base = jnp.where(cid == 0, sid * _K0, _NS * _K0 + sid * _K1)
  start = jnp.minimum(base, _TOTC - _KMAX)
  off = base - start
  pltpu.sync_copy(src_hbm.at[pl.ds(start, _KMAX)], idxs_all)
  pltpu.sync_copy(dst_hbm.at[pl.ds(start, _KMAX)], idxd_all)
  plsc.subcore_barrier()

  def _fire(chunk, rows, sem):
    c = jnp.minimum(chunk, nchunk - 1)
    pltpu.async_copy(h_hbm.at[idxs_all.at[off + c]], rows, sem)

  def _drain(rows, sem):
    pltpu.make_async_copy(h_hbm.at[pl.ds(0, _CR)], rows, sem).wait()

  def _scatter(chunk, rows):
    pltpu.sync_copy(rows, acc.at[idxd_all.at[off + chunk]], add=True)

  _fire(0, r_a, sem_a)

  def _pair(p, carry):
    ca = 2 * p
    cb = ca + 1
    _fire(cb, r_b, sem_b)
    _drain(r_a, sem_a)
    _scatter(ca, r_a)
    _fire(ca + 2, r_a, sem_a)
    _drain(r_b, sem_b)
    _scatter(cb, r_b)
    return carry

  lax.fori_loop(0, nchunk // 2, _pair, 0)
  _drain(r_a, sem_a)
  plsc.subcore_barrier()

  pltpu.sync_copy(acc.at[pl.ds(sid * _ZSL, _ZSL)],
                  out_hbm.at[cid, pl.ds(sid * _ZSL, _ZSL)])


_agg_call = pl.kernel(
    _agg_body,
    out_type=jax.ShapeDtypeStruct((_NC, _ACC_ROWS, _DH), _f32),
    mesh=plsc.VectorSubcoreMesh(core_axis_name="c", subcore_axis_name="s", num_cores=_NC),
    scratch_types=[
        pltpu.VMEM((_KMAX, _CR), jnp.int32),
        pltpu.VMEM((_KMAX, _CR), jnp.int32),
        pltpu.VMEM((_CR, _DH), _f32),
        pltpu.VMEM((_CR, _DH), _f32),
        pltpu.VMEM_SHARED((_ACC_ROWS, _DH), _f32),
        pltpu.SemaphoreType.DMA,
        pltpu.SemaphoreType.DMA,
    ],
    compiler_params=pltpu.CompilerParams(use_tc_tiling_on_sc=False),
)


_GRID = 10
_BR = _N // _GRID


def _norms(deg_ref):
  deg_o = sum(deg_ref[c, 0] for c in range(_NC))
  deg_i = sum(deg_ref[c, 1] for c in range(_NC))
  ns = jnp.where(deg_o > 0, lax.rsqrt(jnp.maximum(deg_o, 1.0)), 1.0)
  nd = jnp.where(deg_i > 0, lax.rsqrt(jnp.maximum(deg_i, 1.0)), 1.0)
  return ns, nd


def _l1_body(x_ref, w_ref, deg_ref, o_ref):
  ns, _ = _norms(deg_ref)
  o_ref[...] = jnp.dot(x_ref[...], w_ref[...], preferred_element_type=_f32) * ns


_l1_call = pl.pallas_call(
    _l1_body,
    grid=(_GRID,),
    in_specs=[
        pl.BlockSpec((_BR, _DIN), lambda i: (i, 0)),
        pl.BlockSpec((_DIN, _DH), lambda i: (0, 0)),
        pl.BlockSpec((_NC, 2, _BR, _DH), lambda i: (0, 0, i, 0)),
    ],
    out_specs=pl.BlockSpec((_BR, _DH), lambda i: (i, 0)),
    out_shape=jax.ShapeDtypeStruct((_TBL_ROWS, _DH), _f32),
)


def _mid_body(a_ref, deg_ref, b_ref, o_ref):
  a = sum(a_ref[c] for c in range(_NC))
  ns, nd = _norms(deg_ref)
  h = jnp.maximum(a * nd + b_ref[...], 0.0)
  o_ref[...] = h * ns


_mid_call = pl.pallas_call(
    _mid_body,
    grid=(_GRID,),
    in_specs=[
        pl.BlockSpec((_NC, _BR, _DH), lambda i: (0, i, 0)),
        pl.BlockSpec((_NC, 2, _BR, _DH), lambda i: (0, 0, i, 0)),
        pl.BlockSpec((1, _DH), lambda i: (0, 0)),
    ],
    out_specs=pl.BlockSpec((_BR, _DH), lambda i: (i, 0)),
    out_shape=jax.ShapeDtypeStruct((_TBL_ROWS, _DH), _f32),
)


def _fin_body(a_ref, deg_ref, w_ref, b_ref, o_ref):
  a = sum(a_ref[c] for c in range(_NC))
  _, nd = _norms(deg_ref)
  o_ref[...] = (
      jnp.dot(a * nd, w_ref[...], preferred_element_type=_f32) + b_ref[...])


_fin_call = pl.pallas_call(
    _fin_body,
    grid=(_GRID,),
    in_specs=[
        pl.BlockSpec((_NC, _BR, _DH), lambda i: (0, i, 0)),
        pl.BlockSpec((_NC, 2, _BR, _DH), lambda i: (0, 0, i, 0)),
        pl.BlockSpec((_DH, _DOUT), lambda i: (0, 0)),
        pl.BlockSpec((1, _DOUT), lambda i: (0, 0)),
    ],
    out_specs=pl.BlockSpec((_BR, _DOUT), lambda i: (i, 0)),
    out_shape=jax.ShapeDtypeStruct((_N, _DOUT), _f32),
)


@jax.jit
def kernel(x, edge_index, W1, b1, W2, b2):
  src = edge_index[0].astype(jnp.int32)
  dst = edge_index[1].astype(jnp.int32)
  pad = _EP - _E
  padv = jnp.full((pad,), _DUMP, jnp.int32)
  srcp = jnp.concatenate([src, padv]).reshape(_EP // _CR, _CR)
  dstp = jnp.concatenate([dst, padv]).reshape(_EP // _CR, _CR)

  zeros_acc = jnp.zeros((_ACC_ROWS, _DH), _f32)
  ones_cr = jnp.ones((_CR, _DH), _f32)

  degp = _deg_call(zeros_acc, ones_cr, srcp, dstp)
  h1t = _l1_call(x, W1, degp)
  a1p = _agg_call(zeros_acc, h1t, srcp, dstp)
  h2t = _mid_call(a1p, degp, b1.reshape(1, _DH))
  a2p = _agg_call(zeros_acc, h2t, srcp, dstp)
  return _fin_call(a2p, degp, W2, b2.reshape(1, _DOUT))


# speedup vs baseline: 1.1921x; 1.1195x over previous
import jax
import jax.numpy as jnp
from jax import lax
from jax.experimental import pallas as pl
from jax.experimental.pallas import tpu as pltpu
from jax.experimental.pallas import tpu_sc as plsc

_N = 10000
_E = 320000
_DIN = 128
_DH = 16
_DOUT = 128

_NC = 2
_NS = 16
_NW = _NC * _NS
_EPW = 10240
_EP = _NW * _EPW
_IDX_W = 128
_ROWS_PW = _EPW // _IDX_W
_CHUNK_ROWS = 16
_NCHUNK = _ROWS_PW // _CHUNK_ROWS
_CR = _CHUNK_ROWS * _IDX_W
_DCR = 8
_NDCHUNK = _ROWS_PW // _DCR
_DUMP = _N
_ACC_ROWS = 10240
_TBL_ROWS = _N + 16
_ZSL = _ACC_ROWS // _NS

_f32 = jnp.float32


def _fill(ref, n, vec):
  def body(i, c):
    ref[i] = vec
    return c

  lax.fori_loop(0, n, body, 0)


def _deg_body(src_hbm, dst_hbm, out_hbm, idxs_all, idxd_all, ones_v, zero_v,
              acc_o, acc_i, sem_o, sem_i):
  cid = lax.axis_index("c")
  sid = lax.axis_index("s")
  wid = cid * _NS + sid

  _fill(zero_v, _ZSL, jnp.zeros((16,), _f32))
  _fill(ones_v, _IDX_W, jnp.ones((16,), _f32))
  pltpu.sync_copy(zero_v, acc_o.at[pl.ds(sid * _ZSL, _ZSL)])
  pltpu.sync_copy(zero_v, acc_i.at[pl.ds(sid * _ZSL, _ZSL)])
  base = wid * _ROWS_PW
  pltpu.sync_copy(src_hbm.at[pl.ds(base, _ROWS_PW)], idxs_all)
  pltpu.sync_copy(dst_hbm.at[pl.ds(base, _ROWS_PW)], idxd_all)
  plsc.subcore_barrier()

  def _chunk(c, carry):
    r0 = c * _DCR
    for j in range(_DCR):
      pltpu.async_copy(ones_v, acc_o.at[idxs_all.at[r0 + j]], sem_o, add=True)
      pltpu.async_copy(ones_v, acc_i.at[idxd_all.at[r0 + j]], sem_i, add=True)
    for j in range(_DCR):
      pltpu.make_async_copy(ones_v, acc_o.at[idxs_all.at[r0 + j]], sem_o).wait()
      pltpu.make_async_copy(ones_v, acc_i.at[idxd_all.at[r0 + j]], sem_i).wait()
    return carry

  lax.fori_loop(0, _NDCHUNK, _chunk, 0)
  plsc.subcore_barrier()

  pltpu.sync_copy(acc_o.at[pl.ds(sid * _ZSL, _ZSL)],
                  out_hbm.at[cid, 0, pl.ds(sid * _ZSL, _ZSL)])
  pltpu.sync_copy(acc_i.at[pl.ds(sid * _ZSL, _ZSL)],
                  out_hbm.at[cid, 1, pl.ds(sid * _ZSL, _ZSL)])


_deg_call = pl.kernel(
    _deg_body,
    out_type=jax.ShapeDtypeStruct((_NC, 2, _ACC_ROWS, _DH), _f32),
    mesh=plsc.VectorSubcoreMesh(core_axis_name="c", subcore_axis_name="s"),
    scratch_types=[
        pltpu.VMEM((_ROWS_PW, _IDX_W), jnp.int32),
        pltpu.VMEM((_ROWS_PW, _IDX_W), jnp.int32),
        pltpu.VMEM((_IDX_W, _DH), _f32),
        pltpu.VMEM((_ZSL, _DH), _f32),
        pltpu.VMEM_SHARED((_ACC_ROWS, _DH), _f32),
        pltpu.VMEM_SHARED((_ACC_ROWS, _DH), _f32),
        pltpu.SemaphoreType.DMA,
        pltpu.SemaphoreType.DMA,
    ],
    compiler_params=pltpu.CompilerParams(use_tc_tiling_on_sc=False),
)


def _agg_body(h_hbm, src_hbm, dst_hbm, out_hbm, idxs_all, idxd_all, r_a, r_b,
              zero_v, acc, sem_a, sem_b):
  cid = lax.axis_index("c")
  sid = lax.axis_index("s")
  wid = cid * _NS + sid

  _fill(zero_v, _ZSL, jnp.zeros((16,), _f32))
  pltpu.sync_copy(zero_v, acc.at[pl.ds(sid * _ZSL, _ZSL)])
  base = wid * _ROWS_PW
  pltpu.sync_copy(src_hbm.at[pl.ds(base, _ROWS_PW)], idxs_all)
  pltpu.sync_copy(dst_hbm.at[pl.ds(base, _ROWS_PW)], idxd_all)
  plsc.subcore_barrier()

  def _fire(chunk, rows, sem):
    for j in range(_CHUNK_ROWS):
      pltpu.async_copy(h_hbm.at[idxs_all.at[chunk * _CHUNK_ROWS + j]],
                       rows.at[pl.ds(j * _IDX_W, _IDX_W)], sem)

  def _drain(rows, sem):
    pltpu.make_async_copy(h_hbm.at[pl.ds(0, _CR)], rows, sem).wait()

  def _scatter(chunk, rows):
    for j in range(_CHUNK_ROWS):
      pltpu.sync_copy(rows.at[pl.ds(j * _IDX_W, _IDX_W)],
                      acc.at[idxd_all.at[chunk * _CHUNK_ROWS + j]], add=True)

  _fire(0, r_a, sem_a)
  _fire(1, r_b, sem_b)
  _drain(r_a, sem_a)
  _scatter(0, r_a)
  _fire(2, r_a, sem_a)
  _drain(r_b, sem_b)
  _scatter(1, r_b)
  _fire(3, r_b, sem_b)
  _drain(r_a, sem_a)
  _scatter(2, r_a)
  _fire(4, r_a, sem_a)
  _drain(r_b, sem_b)
  _scatter(3, r_b)
  _drain(r_a, sem_a)
  _scatter(4, r_a)
  plsc.subcore_barrier()

  pltpu.sync_copy(acc.at[pl.ds(sid * _ZSL, _ZSL)],
                  out_hbm.at[cid, pl.ds(sid * _ZSL, _ZSL)])


_agg_call = pl.kernel(
    _agg_body,
    out_type=jax.ShapeDtypeStruct((_NC, _ACC_ROWS, _DH), _f32),
    mesh=plsc.VectorSubcoreMesh(core_axis_name="c", subcore_axis_name="s"),
    scratch_types=[
        pltpu.VMEM((_ROWS_PW, _IDX_W), jnp.int32),
        pltpu.VMEM((_ROWS_PW, _IDX_W), jnp.int32),
        pltpu.VMEM((_CR, _DH), _f32),
        pltpu.VMEM((_CR, _DH), _f32),
        pltpu.VMEM((_ZSL, _DH), _f32),
        pltpu.VMEM_SHARED((_ACC_ROWS, _DH), _f32),
        pltpu.SemaphoreType.DMA,
        pltpu.SemaphoreType.DMA,
    ],
    compiler_params=pltpu.CompilerParams(use_tc_tiling_on_sc=False),
)


_GRID = 10
_BR = _N // _GRID


def _norms(deg_ref):
  deg_o = sum(deg_ref[c, 0] for c in range(_NC))
  deg_i = sum(deg_ref[c, 1] for c in range(_NC))
  ns = jnp.where(deg_o > 0, lax.rsqrt(jnp.maximum(deg_o, 1.0)), 1.0)
  nd = jnp.where(deg_i > 0, lax.rsqrt(jnp.maximum(deg_i, 1.0)), 1.0)
  return ns, nd


def _l1_body(x_ref, w_ref, deg_ref, o_ref):
  ns, _ = _norms(deg_ref)
  o_ref[...] = jnp.dot(x_ref[...], w_ref[...], preferred_element_type=_f32) * ns


_l1_call = pl.pallas_call(
    _l1_body,
    grid=(_GRID,),
    in_specs=[
        pl.BlockSpec((_BR, _DIN), lambda i: (i, 0)),
        pl.BlockSpec((_DIN, _DH), lambda i: (0, 0)),
        pl.BlockSpec((_NC, 2, _BR, _DH), lambda i: (0, 0, i, 0)),
    ],
    out_specs=pl.BlockSpec((_BR, _DH), lambda i: (i, 0)),
    out_shape=jax.ShapeDtypeStruct((_TBL_ROWS, _DH), _f32),
)


def _mid_body(a_ref, deg_ref, b_ref, o_ref):
  a = sum(a_ref[c] for c in range(_NC))
  ns, nd = _norms(deg_ref)
  h = jnp.maximum(a * nd + b_ref[...], 0.0)
  o_ref[...] = h * ns


_mid_call = pl.pallas_call(
    _mid_body,
    grid=(_GRID,),
    in_specs=[
        pl.BlockSpec((_NC, _BR, _DH), lambda i: (0, i, 0)),
        pl.BlockSpec((_NC, 2, _BR, _DH), lambda i: (0, 0, i, 0)),
        pl.BlockSpec((1, _DH), lambda i: (0, 0)),
    ],
    out_specs=pl.BlockSpec((_BR, _DH), lambda i: (i, 0)),
    out_shape=jax.ShapeDtypeStruct((_TBL_ROWS, _DH), _f32),
)


def _fin_body(a_ref, deg_ref, w_ref, b_ref, o_ref):
  a = sum(a_ref[c] for c in range(_NC))
  _, nd = _norms(deg_ref)
  o_ref[...] = (
      jnp.dot(a * nd, w_ref[...], preferred_element_type=_f32) + b_ref[...])


_fin_call = pl.pallas_call(
    _fin_body,
    grid=(_GRID,),
    in_specs=[
        pl.BlockSpec((_NC, _BR, _DH), lambda i: (0, i, 0)),
        pl.BlockSpec((_NC, 2, _BR, _DH), lambda i: (0, 0, i, 0)),
        pl.BlockSpec((_DH, _DOUT), lambda i: (0, 0)),
        pl.BlockSpec((1, _DOUT), lambda i: (0, 0)),
    ],
    out_specs=pl.BlockSpec((_BR, _DOUT), lambda i: (i, 0)),
    out_shape=jax.ShapeDtypeStruct((_N, _DOUT), _f32),
)


@jax.jit
def kernel(x, edge_index, W1, b1, W2, b2):
  src = edge_index[0].astype(jnp.int32)
  dst = edge_index[1].astype(jnp.int32)
  pad = _EP - _E
  padv = jnp.full((pad,), _DUMP, jnp.int32)
  srcp = jnp.concatenate([src, padv]).reshape(_EP // _IDX_W, _IDX_W)
  dstp = jnp.concatenate([dst, padv]).reshape(_EP // _IDX_W, _IDX_W)

  degp = _deg_call(srcp, dstp)
  h1t = _l1_call(x, W1, degp)
  a1p = _agg_call(h1t, srcp, dstp)
  h2t = _mid_call(a1p, degp, b1.reshape(1, _DH))
  a2p = _agg_call(h2t, srcp, dstp)
  return _fin_call(a2p, degp, W2, b2.reshape(1, _DOUT))


# speedup vs baseline: 1.1996x; 1.0063x over previous
import jax
import jax.numpy as jnp
from jax import lax
from jax.experimental import pallas as pl
from jax.experimental.pallas import tpu as pltpu
from jax.experimental.pallas import tpu_sc as plsc

_N = 10000
_E = 320000
_DIN = 128
_DH = 16
_DOUT = 128

_NC = 2
_NS = 16
_NW = _NC * _NS
_EPW = 10240
_EP = _NW * _EPW
_IDX_W = 128
_ROWS_PW = _EPW // _IDX_W
_CHUNK_ROWS = 16
_NCHUNK = _ROWS_PW // _CHUNK_ROWS
_CR = _CHUNK_ROWS * _IDX_W
_DCR = 8
_NDCHUNK = _ROWS_PW // _DCR
_DUMP = _N
_ACC_ROWS = 10240
_TBL_ROWS = _N + 16
_ZSL = _ACC_ROWS // _NS
_ZQ = _ZSL // 20

_f32 = jnp.float32


def _fill(ref, n, vec):
  def body(i, c):
    ref[i] = vec
    return c

  lax.fori_loop(0, n, body, 0)


def _deg_body(src_hbm, dst_hbm, out_hbm, idxs_all, idxd_all, ones_v, zero_v,
              acc_o, acc_i, sem_o, sem_i):
  cid = lax.axis_index("c")
  sid = lax.axis_index("s")
  wid = cid * _NS + sid

  _fill(zero_v, _ZSL, jnp.zeros((16,), _f32))
  _fill(ones_v, _IDX_W, jnp.ones((16,), _f32))
  pltpu.sync_copy(zero_v, acc_o.at[pl.ds(sid * _ZSL, _ZSL)])
  pltpu.sync_copy(zero_v, acc_i.at[pl.ds(sid * _ZSL, _ZSL)])
  base = wid * _ROWS_PW
  pltpu.sync_copy(src_hbm.at[pl.ds(base, _ROWS_PW)], idxs_all)
  pltpu.sync_copy(dst_hbm.at[pl.ds(base, _ROWS_PW)], idxd_all)
  plsc.subcore_barrier()

  def _chunk(c, carry):
    r0 = c * _DCR
    for j in range(_DCR):
      pltpu.async_copy(ones_v, acc_o.at[idxs_all.at[r0 + j]], sem_o, add=True)
      pltpu.async_copy(ones_v, acc_i.at[idxd_all.at[r0 + j]], sem_i, add=True)

    @pl.when(c > 0)
    def _():
      for j in range(_DCR):
        pltpu.make_async_copy(ones_v, acc_o.at[idxs_all.at[j]], sem_o).wait()
        pltpu.make_async_copy(ones_v, acc_i.at[idxd_all.at[j]], sem_i).wait()

    return carry

  lax.fori_loop(0, _NDCHUNK, _chunk, 0)
  for j in range(_DCR):
    pltpu.make_async_copy(ones_v, acc_o.at[idxs_all.at[j]], sem_o).wait()
    pltpu.make_async_copy(ones_v, acc_i.at[idxd_all.at[j]], sem_i).wait()
  plsc.subcore_barrier()

  pltpu.sync_copy(acc_o.at[pl.ds(sid * _ZSL, _ZSL)],
                  out_hbm.at[cid, 0, pl.ds(sid * _ZSL, _ZSL)])
  pltpu.sync_copy(acc_i.at[pl.ds(sid * _ZSL, _ZSL)],
                  out_hbm.at[cid, 1, pl.ds(sid * _ZSL, _ZSL)])


_deg_call = pl.kernel(
    _deg_body,
    out_type=jax.ShapeDtypeStruct((_NC, 2, _ACC_ROWS, _DH), _f32),
    mesh=plsc.VectorSubcoreMesh(core_axis_name="c", subcore_axis_name="s"),
    scratch_types=[
        pltpu.VMEM((_ROWS_PW, _IDX_W), jnp.int32),
        pltpu.VMEM((_ROWS_PW, _IDX_W), jnp.int32),
        pltpu.VMEM((_IDX_W, _DH), _f32),
        pltpu.VMEM((_ZSL, _DH), _f32),
        pltpu.VMEM_SHARED((_ACC_ROWS, _DH), _f32),
        pltpu.VMEM_SHARED((_ACC_ROWS, _DH), _f32),
        pltpu.SemaphoreType.DMA,
        pltpu.SemaphoreType.DMA,
    ],
    compiler_params=pltpu.CompilerParams(use_tc_tiling_on_sc=False),
)


def _agg_body(h_hbm, src_hbm, dst_hbm, out_hbm, idxs_all, idxd_all, r_a, r_b,
              r_c, zero_v, acc, sem_a, sem_b, sem_c):
  cid = lax.axis_index("c")
  sid = lax.axis_index("s")
  wid = cid * _NS + sid

  _fill(zero_v, _ZQ, jnp.zeros((16,), _f32))
  for k in range(20):
    pltpu.sync_copy(zero_v, acc.at[pl.ds(sid * _ZSL + k * _ZQ, _ZQ)])
  base = wid * _ROWS_PW
  pltpu.sync_copy(src_hbm.at[pl.ds(base, _ROWS_PW)], idxs_all)
  pltpu.sync_copy(dst_hbm.at[pl.ds(base, _ROWS_PW)], idxd_all)
  plsc.subcore_barrier()

  def _fire(chunk, rows, sem):
    for j in range(_CHUNK_ROWS):
      pltpu.async_copy(h_hbm.at[idxs_all.at[chunk * _CHUNK_ROWS + j]],
                       rows.at[pl.ds(j * _IDX_W, _IDX_W)], sem)

  def _drain(rows, sem):
    pltpu.make_async_copy(h_hbm.at[pl.ds(0, _CR)], rows, sem).wait()

  def _scatter(chunk, rows):
    for j in range(_CHUNK_ROWS):
      pltpu.sync_copy(rows.at[pl.ds(j * _IDX_W, _IDX_W)],
                      acc.at[idxd_all.at[chunk * _CHUNK_ROWS + j]], add=True)

  _fire(0, r_a, sem_a)
  _fire(1, r_b, sem_b)
  _drain(r_a, sem_a)
  _scatter(0, r_a)
  _fire(2, r_c, sem_c)
  _drain(r_b, sem_b)
  _scatter(1, r_b)
  _fire(3, r_a, sem_a)
  _drain(r_c, sem_c)
  _scatter(2, r_c)
  _fire(4, r_b, sem_b)
  _drain(r_a, sem_a)
  _scatter(3, r_a)
  _drain(r_b, sem_b)
  _scatter(4, r_b)
  plsc.subcore_barrier()

  pltpu.sync_copy(acc.at[pl.ds(sid * _ZSL, _ZSL)],
                  out_hbm.at[cid, pl.ds(sid * _ZSL, _ZSL)])


_agg_call = pl.kernel(
    _agg_body,
    out_type=jax.ShapeDtypeStruct((_NC, _ACC_ROWS, _DH), _f32),
    mesh=plsc.VectorSubcoreMesh(core_axis_name="c", subcore_axis_name="s"),
    scratch_types=[
        pltpu.VMEM((_ROWS_PW, _IDX_W), jnp.int32),
        pltpu.VMEM((_ROWS_PW, _IDX_W), jnp.int32),
        pltpu.VMEM((_CR, _DH), _f32),
        pltpu.VMEM((_CR, _DH), _f32),
        pltpu.VMEM((_CR, _DH), _f32),
        pltpu.VMEM((_ZQ, _DH), _f32),
        pltpu.VMEM_SHARED((_ACC_ROWS, _DH), _f32),
        pltpu.SemaphoreType.DMA,
        pltpu.SemaphoreType.DMA,
        pltpu.SemaphoreType.DMA,
    ],
    compiler_params=pltpu.CompilerParams(use_tc_tiling_on_sc=False),
)


_GRID = 10
_BR = _N // _GRID


def _norms(deg_ref):
  deg_o = sum(deg_ref[c, 0] for c in range(_NC))
  deg_i = sum(deg_ref[c, 1] for c in range(_NC))
  ns = jnp.where(deg_o > 0, lax.rsqrt(jnp.maximum(deg_o, 1.0)), 1.0)
  nd = jnp.where(deg_i > 0, lax.rsqrt(jnp.maximum(deg_i, 1.0)), 1.0)
  return ns, nd


def _l1_body(x_ref, w_ref, deg_ref, o_ref):
  ns, _ = _norms(deg_ref)
  o_ref[...] = jnp.dot(x_ref[...], w_ref[...], preferred_element_type=_f32) * ns


_l1_call = pl.pallas_call(
    _l1_body,
    grid=(_GRID,),
    in_specs=[
        pl.BlockSpec((_BR, _DIN), lambda i: (i, 0)),
        pl.BlockSpec((_DIN, _DH), lambda i: (0, 0)),
        pl.BlockSpec((_NC, 2, _BR, _DH), lambda i: (0, 0, i, 0)),
    ],
    out_specs=pl.BlockSpec((_BR, _DH), lambda i: (i, 0)),
    out_shape=jax.ShapeDtypeStruct((_TBL_ROWS, _DH), _f32),
)


def _mid_body(a_ref, deg_ref, b_ref, o_ref):
  a = sum(a_ref[c] for c in range(_NC))
  ns, nd = _norms(deg_ref)
  h = jnp.maximum(a * nd + b_ref[...], 0.0)
  o_ref[...] = h * ns


_mid_call = pl.pallas_call(
    _mid_body,
    grid=(_GRID,),
    in_specs=[
        pl.BlockSpec((_NC, _BR, _DH), lambda i: (0, i, 0)),
        pl.BlockSpec((_NC, 2, _BR, _DH), lambda i: (0, 0, i, 0)),
        pl.BlockSpec((1, _DH), lambda i: (0, 0)),
    ],
    out_specs=pl.BlockSpec((_BR, _DH), lambda i: (i, 0)),
    out_shape=jax.ShapeDtypeStruct((_TBL_ROWS, _DH), _f32),
)


def _fin_body(a_ref, deg_ref, w_ref, b_ref, o_ref):
  a = sum(a_ref[c] for c in range(_NC))
  _, nd = _norms(deg_ref)
  o_ref[...] = (
      jnp.dot(a * nd, w_ref[...], preferred_element_type=_f32) + b_ref[...])


_fin_call = pl.pallas_call(
    _fin_body,
    grid=(_GRID,),
    in_specs=[
        pl.BlockSpec((_NC, _BR, _DH), lambda i: (0, i, 0)),
        pl.BlockSpec((_NC, 2, _BR, _DH), lambda i: (0, 0, i, 0)),
        pl.BlockSpec((_DH, _DOUT), lambda i: (0, 0)),
        pl.BlockSpec((1, _DOUT), lambda i: (0, 0)),
    ],
    out_specs=pl.BlockSpec((_BR, _DOUT), lambda i: (i, 0)),
    out_shape=jax.ShapeDtypeStruct((_N, _DOUT), _f32),
)


@jax.jit
def kernel(x, edge_index, W1, b1, W2, b2):
  src = edge_index[0].astype(jnp.int32)
  dst = edge_index[1].astype(jnp.int32)
  pad = _EP - _E
  padv = jnp.full((pad,), _DUMP, jnp.int32)
  srcp = jnp.concatenate([src, padv]).reshape(_EP // _IDX_W, _IDX_W)
  dstp = jnp.concatenate([dst, padv]).reshape(_EP // _IDX_W, _IDX_W)

  degp = _deg_call(srcp, dstp)
  h1t = _l1_call(x, W1, degp)
  a1p = _agg_call(h1t, srcp, dstp)
  h2t = _mid_call(a1p, degp, b1.reshape(1, _DH))
  a2p = _agg_call(h2t, srcp, dstp)
  return _fin_call(a2p, degp, W2, b2.reshape(1, _DOUT))


# speedup vs baseline: 1.2079x; 1.0069x over previous
import jax
import jax.numpy as jnp
from jax import lax
from jax.experimental import pallas as pl
from jax.experimental.pallas import tpu as pltpu
from jax.experimental.pallas import tpu_sc as plsc

_N = 10000
_E = 320000
_DIN = 128
_DH = 16
_DOUT = 128

_NC = 2
_NS = 16
_NW = _NC * _NS
_EPW = 10240
_EP = _NW * _EPW
_IDX_W = 128
_CR = 2048
_NCHUNK = _EPW // _CR
_GPC = _CR // _IDX_W
_DUMP = _N
_ACC_ROWS = 10240
_TBL_ROWS = _N + 16
_ZSL = _ACC_ROWS // _NS
_ZQ = _ZSL // 20

_f32 = jnp.float32


def _fill(ref, n, vec):
  def body(i, c):
    ref[i] = vec
    return c

  lax.fori_loop(0, n, body, 0)


def _deg_body(src_hbm, dst_hbm, out_hbm, idxs_all, idxd_all, ones_v, zero_v,
              acc_o, acc_i, sem_o, sem_i):
  cid = lax.axis_index("c")
  sid = lax.axis_index("s")
  wid = cid * _NS + sid

  _fill(zero_v, _ZSL, jnp.zeros((16,), _f32))
  _fill(ones_v, _IDX_W, jnp.ones((16,), _f32))
  pltpu.sync_copy(zero_v, acc_o.at[pl.ds(sid * _ZSL, _ZSL)])
  pltpu.sync_copy(zero_v, acc_i.at[pl.ds(sid * _ZSL, _ZSL)])
  base = wid * _NCHUNK
  pltpu.sync_copy(src_hbm.at[pl.ds(base, _NCHUNK)], idxs_all)
  pltpu.sync_copy(dst_hbm.at[pl.ds(base, _NCHUNK)], idxd_all)
  plsc.subcore_barrier()

  for c in range(_NCHUNK):
    for j in range(_GPC):
      pltpu.async_copy(ones_v, acc_o.at[idxs_all.at[c, pl.ds(j * _IDX_W, _IDX_W)]],
                       sem_o, add=True)
      pltpu.async_copy(ones_v, acc_i.at[idxd_all.at[c, pl.ds(j * _IDX_W, _IDX_W)]],
                       sem_i, add=True)
    if c > 0:
      for j in range(_GPC):
        pltpu.make_async_copy(ones_v, acc_o.at[idxs_all.at[0, pl.ds(0, _IDX_W)]],
                              sem_o).wait()
        pltpu.make_async_copy(ones_v, acc_i.at[idxd_all.at[0, pl.ds(0, _IDX_W)]],
                              sem_i).wait()
  for j in range(_GPC):
    pltpu.make_async_copy(ones_v, acc_o.at[idxs_all.at[0, pl.ds(0, _IDX_W)]],
                          sem_o).wait()
    pltpu.make_async_copy(ones_v, acc_i.at[idxd_all.at[0, pl.ds(0, _IDX_W)]],
                          sem_i).wait()
  plsc.subcore_barrier()

  pltpu.sync_copy(acc_o.at[pl.ds(sid * _ZSL, _ZSL)],
                  out_hbm.at[cid, 0, pl.ds(sid * _ZSL, _ZSL)])
  pltpu.sync_copy(acc_i.at[pl.ds(sid * _ZSL, _ZSL)],
                  out_hbm.at[cid, 1, pl.ds(sid * _ZSL, _ZSL)])


_deg_call = pl.kernel(
    _deg_body,
    out_type=jax.ShapeDtypeStruct((_NC, 2, _ACC_ROWS, _DH), _f32),
    mesh=plsc.VectorSubcoreMesh(core_axis_name="c", subcore_axis_name="s"),
    scratch_types=[
        pltpu.VMEM((_NCHUNK, _CR), jnp.int32),
        pltpu.VMEM((_NCHUNK, _CR), jnp.int32),
        pltpu.VMEM((_IDX_W, _DH), _f32),
        pltpu.VMEM((_ZSL, _DH), _f32),
        pltpu.VMEM_SHARED((_ACC_ROWS, _DH), _f32),
        pltpu.VMEM_SHARED((_ACC_ROWS, _DH), _f32),
        pltpu.SemaphoreType.DMA,
        pltpu.SemaphoreType.DMA,
    ],
    compiler_params=pltpu.CompilerParams(use_tc_tiling_on_sc=False),
)


def _agg_body(h_hbm, src_hbm, dst_hbm, out_hbm, idxs_all, idxd_all, r_a, r_b,
              r_c, zero_v, acc, sem_a, sem_b, sem_c, sem_s):
  cid = lax.axis_index("c")
  sid = lax.axis_index("s")
  wid = cid * _NS + sid

  _fill(zero_v, _ZQ, jnp.zeros((16,), _f32))
  for k in range(20):
    pltpu.sync_copy(zero_v, acc.at[pl.ds(sid * _ZSL + k * _ZQ, _ZQ)])
  base = wid * _NCHUNK
  pltpu.sync_copy(src_hbm.at[pl.ds(base, _NCHUNK)], idxs_all)
  pltpu.sync_copy(dst_hbm.at[pl.ds(base, _NCHUNK)], idxd_all)
  plsc.subcore_barrier()

  def _fire(chunk, rows, sem):
    for j in range(_GPC):
      pltpu.async_copy(
          h_hbm.at[idxs_all.at[chunk, pl.ds(j * _IDX_W, _IDX_W)]],
          rows.at[pl.ds(j * _IDX_W, _IDX_W)], sem)

  def _drain(rows, sem):
    pltpu.make_async_copy(h_hbm.at[pl.ds(0, _CR)], rows, sem).wait()

  def _scatter(chunk, rows):
    pltpu.async_copy(rows, acc.at[idxd_all.at[chunk]], sem_s, add=True)

  def _drain_s(rows):
    pltpu.make_async_copy(rows, acc.at[idxd_all.at[0]], sem_s).wait()

  _fire(0, r_a, sem_a)
  _fire(1, r_b, sem_b)
  _drain(r_a, sem_a)
  _scatter(0, r_a)
  _fire(2, r_c, sem_c)
  _drain(r_b, sem_b)
  _scatter(1, r_b)
  _drain_s(r_a)
  _fire(3, r_a, sem_a)
  _drain(r_c, sem_c)
  _scatter(2, r_c)
  _drain_s(r_b)
  _fire(4, r_b, sem_b)
  _drain(r_a, sem_a)
  _scatter(3, r_a)
  _drain(r_b, sem_b)
  _scatter(4, r_b)
  _drain_s(r_c)
  _drain_s(r_a)
  _drain_s(r_b)
  plsc.subcore_barrier()

  pltpu.sync_copy(acc.at[pl.ds(sid * _ZSL, _ZSL)],
                  out_hbm.at[cid, pl.ds(sid * _ZSL, _ZSL)])


_agg_call = pl.kernel(
    _agg_body,
    out_type=jax.ShapeDtypeStruct((_NC, _ACC_ROWS, _DH), _f32),
    mesh=plsc.VectorSubcoreMesh(core_axis_name="c", subcore_axis_name="s"),
    scratch_types=[
        pltpu.VMEM((_NCHUNK, _CR), jnp.int32),
        pltpu.VMEM((_NCHUNK, _CR), jnp.int32),
        pltpu.VMEM((_CR, _DH), _f32),
        pltpu.VMEM((_CR, _DH), _f32),
        pltpu.VMEM((_CR, _DH), _f32),
        pltpu.VMEM((_ZQ, _DH), _f32),
        pltpu.VMEM_SHARED((_ACC_ROWS, _DH), _f32),
        pltpu.SemaphoreType.DMA,
        pltpu.SemaphoreType.DMA,
        pltpu.SemaphoreType.DMA,
        pltpu.SemaphoreType.DMA,
    ],
    compiler_params=pltpu.CompilerParams(use_tc_tiling_on_sc=False),
)


_GRID = 10
_BR = _N // _GRID


def _norms(deg_ref):
  deg_o = sum(deg_ref[c, 0] for c in range(_NC))
  deg_i = sum(deg_ref[c, 1] for c in range(_NC))
  ns = jnp.where(deg_o > 0, lax.rsqrt(jnp.maximum(deg_o, 1.0)), 1.0)
  nd = jnp.where(deg_i > 0, lax.rsqrt(jnp.maximum(deg_i, 1.0)), 1.0)
  return ns, nd


def _l1_body(x_ref, w_ref, deg_ref, o_ref):
  ns, _ = _norms(deg_ref)
  o_ref[...] = jnp.dot(x_ref[...], w_ref[...], preferred_element_type=_f32) * ns


_l1_call = pl.pallas_call(
    _l1_body,
    grid=(_GRID,),
    in_specs=[
        pl.BlockSpec((_BR, _DIN), lambda i: (i, 0)),
        pl.BlockSpec((_DIN, _DH), lambda i: (0, 0)),
        pl.BlockSpec((_NC, 2, _BR, _DH), lambda i: (0, 0, i, 0)),
    ],
    out_specs=pl.BlockSpec((_BR, _DH), lambda i: (i, 0)),
    out_shape=jax.ShapeDtypeStruct((_TBL_ROWS, _DH), _f32),
)


def _mid_body(a_ref, deg_ref, b_ref, o_ref):
  a = sum(a_ref[c] for c in range(_NC))
  ns, nd = _norms(deg_ref)
  h = jnp.maximum(a * nd + b_ref[...], 0.0)
  o_ref[...] = h * ns


_mid_call = pl.pallas_call(
    _mid_body,
    grid=(_GRID,),
    in_specs=[
        pl.BlockSpec((_NC, _BR, _DH), lambda i: (0, i, 0)),
        pl.BlockSpec((_NC, 2, _BR, _DH), lambda i: (0, 0, i, 0)),
        pl.BlockSpec((1, _DH), lambda i: (0, 0)),
    ],
    out_specs=pl.BlockSpec((_BR, _DH), lambda i: (i, 0)),
    out_shape=jax.ShapeDtypeStruct((_TBL_ROWS, _DH), _f32),
)


def _fin_body(a_ref, deg_ref, w_ref, b_ref, o_ref):
  a = sum(a_ref[c] for c in range(_NC))
  _, nd = _norms(deg_ref)
  o_ref[...] = (
      jnp.dot(a * nd, w_ref[...], preferred_element_type=_f32) + b_ref[...])


_fin_call = pl.pallas_call(
    _fin_body,
    grid=(_GRID,),
    in_specs=[
        pl.BlockSpec((_NC, _BR, _DH), lambda i: (0, i, 0)),
        pl.BlockSpec((_NC, 2, _BR, _DH), lambda i: (0, 0, i, 0)),
        pl.BlockSpec((_DH, _DOUT), lambda i: (0, 0)),
        pl.BlockSpec((1, _DOUT), lambda i: (0, 0)),
    ],
    out_specs=pl.BlockSpec((_BR, _DOUT), lambda i: (i, 0)),
    out_shape=jax.ShapeDtypeStruct((_N, _DOUT), _f32),
)


@jax.jit
def kernel(x, edge_index, W1, b1, W2, b2):
  src = edge_index[0].astype(jnp.int32)
  dst = edge_index[1].astype(jnp.int32)
  pad = _EP - _E
  padv = jnp.full((pad,), _DUMP, jnp.int32)
  srcp = jnp.concatenate([src, padv]).reshape(_EP // _CR, _CR)
  dstp = jnp.concatenate([dst, padv]).reshape(_EP // _CR, _CR)

  degp = _deg_call(srcp, dstp)
  h1t = _l1_call(x, W1, degp)
  a1p = _agg_call(h1t, srcp, dstp)
  h2t = _mid_call(a1p, degp, b1.reshape(1, _DH))
  a2p = _agg_call(h2t, srcp, dstp)
  return _fin_call(a2p, degp, W2, b2.reshape(1, _DOUT))
